# baseline (speedup 1.0000x reference)
import jax
import jax.numpy as jnp
from jax.experimental import pallas as pl
from jax.experimental.pallas import tpu as pltpu

_HID = 4096
_NE = 64
_K = 8
_BT = 1024
_HALF = _BT // 2


def _top8(logits):
    cur = logits.T
    row = jax.lax.broadcasted_iota(jnp.int32, cur.shape, 0)
    vals = []
    idxs = []
    for _ in range(_K):
        m = jnp.max(cur, axis=0, keepdims=True)
        idx = jnp.min(jnp.where(cur == m, row, _NE), axis=0, keepdims=True)
        vals.append(m)
        idxs.append(idx)
        cur = jnp.where(row == idx, -jnp.inf, cur)
    v = jnp.concatenate(vals, axis=0)
    i = jnp.concatenate(idxs, axis=0)
    e = jnp.exp(v - v[:1])
    w = e / jnp.sum(e, axis=0, keepdims=True)
    return w.T, i.T


def _router_block(xa_ref, xb_ref, wt_ref, rw_ref, se_ref):
    wt = wt_ref[...]
    la = jnp.dot(xa_ref[...], wt, preferred_element_type=jnp.float32)
    lb = jnp.dot(xb_ref[...], wt, preferred_element_type=jnp.float32)
    wa, ia = _top8(la)
    wb, ib = _top8(lb)
    rw_ref[:_HALF] = wa
    se_ref[:_HALF] = ia
    rw_ref[_HALF:] = wb
    se_ref[_HALF:] = ib


def kernel(hidden_states, gate_w):
    flat = hidden_states.reshape(-1, _HID)
    n_tok = flat.shape[0]
    wt = gate_w.T
    grid = (n_tok // _BT,)
    rw, se = pl.pallas_call(
        _router_block,
        grid=grid,
        in_specs=[
            pl.BlockSpec((_HALF, _HID), lambda i: (2 * i, 0)),
            pl.BlockSpec((_HALF, _HID), lambda i: (2 * i + 1, 0)),
            pl.BlockSpec((_HID, _NE), lambda i: (0, 0)),
        ],
        out_specs=[
            pl.BlockSpec((_BT, _K), lambda i: (i, 0)),
            pl.BlockSpec((_BT, _K), lambda i: (i, 0)),
        ],
        out_shape=[
            jax.ShapeDtypeStruct((n_tok, _K), jnp.float32),
            jax.ShapeDtypeStruct((n_tok, _K), jnp.int32),
        ],
    )(flat, flat, wt)
    return (rw, se)
